# baseline (speedup 1.0000x reference)
import functools
import jax
import jax.numpy as jnp
from jax import lax
from jax.experimental import pallas as pl
from jax.experimental.pallas import tpu as pltpu
from jax.experimental.pallas import tpu_sc as plsc

_B, _S, _D = 4, 8192, 768
_NC, _NS = 2, 16
_NW = _NC * _NS
_SPW = _S // _NW
_R = 64
_CH = _SPW // _R
_CW = _R * _D


def _sc_add(x_hbm, t_hbm, o_hbm, xbuf, tbuf):
    wid = lax.axis_index("s") * _NC + lax.axis_index("c")
    s_base = wid * _SPW

    def chunk_body(c, carry):
        t_off = (s_base + c * _R) * _D
        pltpu.sync_copy(t_hbm.at[pl.ds(t_off, _CW)], tbuf)

        def batch_body(b, carry2):
            x_off = (b * _S + s_base + c * _R) * _D
            pltpu.sync_copy(x_hbm.at[pl.ds(x_off, _CW)], xbuf)

            def add_body(j, carry3):
                base = j * 128
                for u in range(8):
                    k0 = base + u * 16
                    plsc.addupdate(xbuf.at[pl.ds(k0, 16)], tbuf[pl.ds(k0, 16)])
                return carry3

            lax.fori_loop(0, _CW // 128, add_body, 0)
            pltpu.sync_copy(xbuf, o_hbm.at[pl.ds(x_off, _CW)])
            return carry2

        lax.fori_loop(0, _B, batch_body, 0)
        return carry

    lax.fori_loop(0, _CH, chunk_body, 0)


def kernel(x, pos_table):
    B, S, D = x.shape
    xf = x.reshape(B * S * D)
    tf = pos_table.reshape(S * D)
    mesh = plsc.VectorSubcoreMesh(core_axis_name="c", subcore_axis_name="s")
    run = pl.kernel(
        _sc_add,
        out_type=jax.ShapeDtypeStruct((B * S * D,), jnp.float32),
        mesh=mesh,
        scratch_types=[
            pltpu.VMEM((_CW,), jnp.float32),
            pltpu.VMEM((_CW,), jnp.float32),
        ],
    )
    return run(xf, tf).reshape(B, S, D)


# speedup vs baseline: 1.1882x; 1.1882x over previous
import functools
import jax
import jax.numpy as jnp
from jax import lax
from jax.experimental import pallas as pl
from jax.experimental.pallas import tpu as pltpu
from jax.experimental.pallas import tpu_sc as plsc

_B, _S, _D = 4, 8192, 768
_NC, _NS = 2, 16
_NW = _NC * _NS
_SPW = _S // _NW
_R = 16
_CW = _R * _D
_CH = _SPW // _R
_U = _CH * _B


def _sc_add(x_hbm, t_hbm, o_hbm,
            in0, in1, in2, out0, out1, out2, tb0, tb1, lsem, ssem, tsem):
    inb = [in0, in1, in2]
    outb = [out0, out1, out2]
    tb = [tb0, tb1]
    wid = lax.axis_index("s") * _NC + lax.axis_index("c")
    s_base = wid * _SPW

    def xoff(i):
        c, b = divmod(i, _B)
        return (b * _S + s_base + c * _R) * _D

    def toff(c):
        return (s_base + c * _R) * _D

    tdesc = {}
    for c in range(min(2, _CH)):
        tdesc[c] = pltpu.async_copy(
            t_hbm.at[pl.ds(toff(c), _CW)], tb[c % 2], tsem.at[c % 2])
    ldesc = {}
    for i in range(min(3, _U)):
        ldesc[i] = pltpu.async_copy(
            x_hbm.at[pl.ds(xoff(i), _CW)], inb[i % 3], lsem.at[i % 3])
    sdesc = {}
    for i in range(_U):
        c, b = divmod(i, _B)
        bi = i % 3
        tc = c % 2
        if b == 0:
            tdesc[c].wait()
        ldesc[i].wait()
        if i >= 3:
            sdesc[i - 3].wait()

        def add_body(j, carry, bi=bi, tc=tc):
            base = j * 128
            for u in range(8):
                sl = pl.ds(base + u * 16, 16)
                outb[bi][sl] = inb[bi][sl] + tb[tc][sl]
            return carry

        lax.fori_loop(0, _CW // 128, add_body, 0)
        sdesc[i] = pltpu.async_copy(
            outb[bi], o_hbm.at[pl.ds(xoff(i), _CW)], ssem.at[bi])
        if i + 3 < _U:
            ldesc[i + 3] = pltpu.async_copy(
                x_hbm.at[pl.ds(xoff(i + 3), _CW)],
                inb[(i + 3) % 3], lsem.at[(i + 3) % 3])
        if b == _B - 1 and c + 2 < _CH:
            tdesc[c + 2] = pltpu.async_copy(
                t_hbm.at[pl.ds(toff(c + 2), _CW)], tb[tc], tsem.at[tc])
    for i in range(max(0, _U - 3), _U):
        sdesc[i].wait()


def kernel(x, pos_table):
    B, S, D = x.shape
    xf = x.reshape(B * S * D)
    tf = pos_table.reshape(S * D)
    mesh = plsc.VectorSubcoreMesh(core_axis_name="c", subcore_axis_name="s")
    run = pl.kernel(
        _sc_add,
        out_type=jax.ShapeDtypeStruct((B * S * D,), jnp.float32),
        mesh=mesh,
        scratch_types=(
            [pltpu.VMEM((_CW,), jnp.float32)] * 6
            + [pltpu.VMEM((_CW,), jnp.float32)] * 2
            + [
                pltpu.SemaphoreType.DMA((3,)),
                pltpu.SemaphoreType.DMA((3,)),
                pltpu.SemaphoreType.DMA((2,)),
            ]
        ),
    )
    return run(xf, tf).reshape(B, S, D)


# speedup vs baseline: 5.4352x; 4.5745x over previous
import jax
import jax.numpy as jnp
from jax.experimental import pallas as pl


def _add_kernel(x_ref, t_ref, o_ref):
    o_ref[...] = x_ref[...] + t_ref[...][None, :, :]


def kernel(x, pos_table):
    B, S, D = x.shape
    S_BLK = 512
    grid = (S // S_BLK,)
    return pl.pallas_call(
        _add_kernel,
        grid=grid,
        in_specs=[
            pl.BlockSpec((B, S_BLK, D), lambda i: (0, i, 0)),
            pl.BlockSpec((S_BLK, D), lambda i: (i, 0)),
        ],
        out_specs=pl.BlockSpec((B, S_BLK, D), lambda i: (0, i, 0)),
        out_shape=jax.ShapeDtypeStruct((B, S, D), x.dtype),
    )(x, pos_table)
